# Initial kernel scaffold; baseline (speedup 1.0000x reference)
import jax
import jax.numpy as jnp
from jax.experimental import pallas as pl


def kernel(exercises, categories, exercise_table, category_table, position_table):
    raise NotImplementedError("write your pallas kernel here")


# speedup vs baseline: 3.6670x; 3.6670x over previous
import functools

import jax
import jax.numpy as jnp
from jax import lax
from jax.experimental import pallas as pl
from jax.experimental.pallas import tpu as pltpu
from jax.experimental.pallas import tpu_sc as plsc

N_DIMS = 64
SEQ_LEN = 200
BATCH = 4096

_INFO = plsc.get_sparse_core_info()
_NC = _INFO.num_cores
_NS = _INFO.num_subcores
_NW = _NC * _NS

_ROWS = BATCH * SEQ_LEN
_ROWS_PER_W = _ROWS // _NW
_K = 128
_NCHUNK = _ROWS_PER_W // _K


def _body(ex_hbm, cat_hbm, etab_hbm, ctab_hbm, ptab_hbm, out_hbm,
          p_v, ie_v, ic_v, be_v, bc_v, sem_e, sem_c):
    wid = lax.axis_index("s") * _NC + lax.axis_index("c")
    w_base = wid * _ROWS_PER_W

    pltpu.sync_copy(ptab_hbm, p_v)

    def chunk_body(ci, carry):
        base = w_base + ci * _K
        pltpu.sync_copy(ex_hbm.at[pl.ds(base, _K)], ie_v)
        pltpu.sync_copy(cat_hbm.at[pl.ds(base, _K)], ic_v)
        ge = pltpu.async_copy(etab_hbm.at[ie_v], be_v, sem_e)
        gc = pltpu.async_copy(ctab_hbm.at[ic_v], bc_v, sem_c)
        ge.wait()
        gc.wait()

        def row_body(r, carry2):
            s = lax.rem(base + r, SEQ_LEN)
            for d in range(N_DIMS // 16):
                sl = pl.ds(d * 16, 16)
                acc = be_v[r, sl] + bc_v[r, sl] + p_v[s, sl]
                be_v[r, sl] = acc
            return carry2

        lax.fori_loop(0, _K, row_body, 0, unroll=2)
        pltpu.sync_copy(be_v, out_hbm.at[pl.ds(base, _K)])
        return carry

    lax.fori_loop(0, _NCHUNK, chunk_body, 0)


@jax.jit
def _run(ex_flat, cat_flat, etab, ctab, ptab):
    mesh = plsc.VectorSubcoreMesh(core_axis_name="c", subcore_axis_name="s")
    f = pl.kernel(
        _body,
        out_type=jax.ShapeDtypeStruct((_ROWS, N_DIMS), jnp.float32),
        mesh=mesh,
        scratch_types=[
            pltpu.VMEM((SEQ_LEN, N_DIMS), jnp.float32),
            pltpu.VMEM((_K,), jnp.int32),
            pltpu.VMEM((_K,), jnp.int32),
            pltpu.VMEM((_K, N_DIMS), jnp.float32),
            pltpu.VMEM((_K, N_DIMS), jnp.float32),
            pltpu.SemaphoreType.DMA,
            pltpu.SemaphoreType.DMA,
        ],
        compiler_params=pltpu.CompilerParams(use_tc_tiling_on_sc=False),
    )
    return f(ex_flat, cat_flat, etab, ctab, ptab)


def kernel(exercises, categories, exercise_table, category_table, position_table):
    ex_flat = exercises.reshape(-1).astype(jnp.int32)
    cat_flat = categories.reshape(-1).astype(jnp.int32)
    out = _run(ex_flat, cat_flat, exercise_table, category_table, position_table)
    return out.reshape(BATCH, SEQ_LEN, N_DIMS)


# speedup vs baseline: 5.0029x; 1.3643x over previous
import jax
import jax.numpy as jnp
from jax import lax
from jax.experimental import pallas as pl
from jax.experimental.pallas import tpu as pltpu
from jax.experimental.pallas import tpu_sc as plsc

N_DIMS = 64
SEQ_LEN = 200
BATCH = 4096

_INFO = plsc.get_sparse_core_info()
_NC = _INFO.num_cores
_NS = _INFO.num_subcores
_NW = _NC * _NS

_ROWS = BATCH * SEQ_LEN
_ROWS_PER_W = _ROWS // _NW
_K = 128
_NCHUNK = _ROWS_PER_W // _K


def _body(ex_hbm, cat_hbm, etab_hbm, ctab_hbm, ptab_hbm, out_hbm,
          p_v, ie0, ie1, ic0, ic1, be0, be1, bc0, bc1,
          si0, si1, se0, se1, sc0, sc1, so0, so1):
    ie = (ie0, ie1)
    ic = (ic0, ic1)
    be = (be0, be1)
    bc = (bc0, bc1)
    si = (si0, si1)
    se = (se0, se1)
    sc = (sc0, sc1)
    so = (so0, so1)

    wid = lax.axis_index("s") * _NC + lax.axis_index("c")
    w_base = wid * _ROWS_PER_W

    pltpu.sync_copy(ptab_hbm, p_v)

    def issue_idx(ci, b):
        base = w_base + ci * _K
        pltpu.async_copy(ex_hbm.at[pl.ds(base, _K)], ie[b], si[b])
        pltpu.async_copy(cat_hbm.at[pl.ds(base, _K)], ic[b], si[b])

    def wait_idx(ci, b):
        base = w_base + ci * _K
        pltpu.make_async_copy(ex_hbm.at[pl.ds(base, _K)], ie[b], si[b]).wait()
        pltpu.make_async_copy(cat_hbm.at[pl.ds(base, _K)], ic[b], si[b]).wait()

    def issue_gathers(b):
        pltpu.async_copy(etab_hbm.at[ie[b]], be[b], se[b])
        pltpu.async_copy(ctab_hbm.at[ic[b]], bc[b], sc[b])

    def wait_gathers(b):
        pltpu.make_async_copy(etab_hbm.at[ie[b]], be[b], se[b]).wait()
        pltpu.make_async_copy(ctab_hbm.at[ic[b]], bc[b], sc[b]).wait()

    def wait_writeback(ci, b):
        base = w_base + ci * _K
        pltpu.make_async_copy(be[b], out_hbm.at[pl.ds(base, _K)], so[b]).wait()

    issue_idx(0, 0)
    issue_idx(1, 1)
    wait_idx(0, 0)
    issue_gathers(0)

    def chunk(ci, b):
        nb = 1 - b
        nci = ci + 1
        wait_gathers(b)

        @pl.when(ci + 2 < _NCHUNK)
        def _():
            issue_idx(ci + 2, b)

        @pl.when(nci < _NCHUNK)
        def _():
            @pl.when(ci >= 1)
            def _():
                wait_writeback(ci - 1, nb)
            wait_idx(nci, nb)
            issue_gathers(nb)

        base = w_base + ci * _K

        def row_body(r, carry2):
            s = lax.rem(base + r, SEQ_LEN)
            for d in range(N_DIMS // 16):
                sl = pl.ds(d * 16, 16)
                be[b][r, sl] = be[b][r, sl] + bc[b][r, sl] + p_v[s, sl]
            return carry2

        lax.fori_loop(0, _K, row_body, 0, unroll=2)
        pltpu.async_copy(be[b], out_hbm.at[pl.ds(base, _K)], so[b])

    def outer(g2, carry):
        for b in range(2):
            chunk(g2 * 2 + b, b)
        return carry

    lax.fori_loop(0, _NCHUNK // 2, outer, 0)

    wait_writeback(_NCHUNK - 2, 0)
    wait_writeback(_NCHUNK - 1, 1)


@jax.jit
def _run(ex_flat, cat_flat, etab, ctab, ptab):
    mesh = plsc.VectorSubcoreMesh(core_axis_name="c", subcore_axis_name="s")
    f = pl.kernel(
        _body,
        out_type=jax.ShapeDtypeStruct((_ROWS, N_DIMS), jnp.float32),
        mesh=mesh,
        scratch_types=[
            pltpu.VMEM((SEQ_LEN, N_DIMS), jnp.float32),
            pltpu.VMEM((_K,), jnp.int32),
            pltpu.VMEM((_K,), jnp.int32),
            pltpu.VMEM((_K,), jnp.int32),
            pltpu.VMEM((_K,), jnp.int32),
            pltpu.VMEM((_K, N_DIMS), jnp.float32),
            pltpu.VMEM((_K, N_DIMS), jnp.float32),
            pltpu.VMEM((_K, N_DIMS), jnp.float32),
            pltpu.VMEM((_K, N_DIMS), jnp.float32),
            pltpu.SemaphoreType.DMA,
            pltpu.SemaphoreType.DMA,
            pltpu.SemaphoreType.DMA,
            pltpu.SemaphoreType.DMA,
            pltpu.SemaphoreType.DMA,
            pltpu.SemaphoreType.DMA,
            pltpu.SemaphoreType.DMA,
            pltpu.SemaphoreType.DMA,
        ],
        compiler_params=pltpu.CompilerParams(use_tc_tiling_on_sc=False),
    )
    return f(ex_flat, cat_flat, etab, ctab, ptab)


def kernel(exercises, categories, exercise_table, category_table, position_table):
    ex_flat = exercises.reshape(-1).astype(jnp.int32)
    cat_flat = categories.reshape(-1).astype(jnp.int32)
    out = _run(ex_flat, cat_flat, exercise_table, category_table, position_table)
    return out.reshape(BATCH, SEQ_LEN, N_DIMS)
